# baseline (speedup 1.0000x reference)
import jax
import jax.numpy as jnp
from jax import lax
from jax.experimental import pallas as pl
from jax.experimental.pallas import tpu as pltpu
from jax.experimental.pallas import tpu_sc as plsc

_BATCH = 16384
_K = 16
_NW = 32
_BPW = _BATCH // _NW
_NGRP = _BPW // _K
_NROW = 2 * _NGRP

_NUSER = 1000000


def _mf_body(wf_hbm, hf_hbm, uidx_hbm, vidx_hbm, out_hbm,
             uidx_v, vidx_v, ugat_v, vgat_v, uadr_v, vadr_v, out_v,
             usem, vsem):
  wid = lax.axis_index("s") * 2 + lax.axis_index("c")
  base = wid * _BPW

  pltpu.sync_copy(uidx_hbm.at[pl.ds(base, _BPW)], uidx_v)
  pltpu.sync_copy(vidx_hbm.at[pl.ds(base, _BPW)], vidx_v)

  def build(g, carry):
    row0 = pl.multiple_of(g * _K, _K)
    u = uidx_v[pl.ds(row0, _K)]
    v = vidx_v[pl.ds(row0, _K)]
    for c in range(_K):
      r = 2 * g + (c // 8)
      s = (c % 8) * _K
      uadr_v[r, pl.ds(s, _K)] = u + (c * _NUSER)
      vadr_v[r, pl.ds(s, _K)] = v + (c * _NUSER)
    return carry

  lax.fori_loop(0, _NGRP, build, 0)

  copies = []
  for r in range(_NROW):
    copies.append(pltpu.async_copy(
        wf_hbm.at[uadr_v.at[r]], ugat_v.at[r], usem))
    copies.append(pltpu.async_copy(
        hf_hbm.at[vadr_v.at[r]], vgat_v.at[r], vsem))
  for cp in copies:
    cp.wait()

  def dot(g, carry):
    row0 = pl.multiple_of(g * _K, _K)
    acc = jnp.zeros((_K,), jnp.float32)
    for c in range(_K):
      r = 2 * g + (c // 8)
      s = (c % 8) * _K
      acc = acc + ugat_v[r, pl.ds(s, _K)] * vgat_v[r, pl.ds(s, _K)]
    out_v[pl.ds(row0, _K)] = acc
    return carry

  lax.fori_loop(0, _NGRP, dot, 0)

  pltpu.sync_copy(out_v, out_hbm.at[pl.ds(base, _BPW)])


@jax.jit
def kernel(x, W, H):
  uidx = x[:, 0].astype(jnp.int32)
  vidx = x[:, 1].astype(jnp.int32)

  mf = pl.kernel(
      _mf_body,
      out_type=jax.ShapeDtypeStruct((_BATCH,), jnp.float32),
      mesh=plsc.VectorSubcoreMesh(core_axis_name="c", subcore_axis_name="s",
                                  num_cores=2, num_subcores=16),
      compiler_params=pltpu.CompilerParams(needs_layout_passes=False),
      scratch_types=[
          pltpu.VMEM((_BPW,), jnp.int32),
          pltpu.VMEM((_BPW,), jnp.int32),
          pltpu.VMEM((_NROW, 128), jnp.float32),
          pltpu.VMEM((_NROW, 128), jnp.float32),
          pltpu.VMEM((_NROW, 128), jnp.int32),
          pltpu.VMEM((_NROW, 128), jnp.int32),
          pltpu.VMEM((_BPW,), jnp.float32),
          pltpu.SemaphoreType.DMA,
          pltpu.SemaphoreType.DMA,
      ],
  )
  return mf(W.T.reshape(-1), H.T.reshape(-1), uidx, vidx)


# speedup vs baseline: 1.0018x; 1.0018x over previous
import jax
import jax.numpy as jnp
from jax import lax
from jax.experimental import pallas as pl
from jax.experimental.pallas import tpu as pltpu
from jax.experimental.pallas import tpu_sc as plsc

_BATCH = 16384
_K = 16
_NW = 32
_BPW = _BATCH // _NW
_NGRP = _BPW // _K
_NROW = 2 * _NGRP

_NUSER = 1000000


def _mf_body(wt_hbm, ht_hbm, wf_hbm, hf_hbm, uidx_hbm, vidx_hbm, out_hbm,
             uidx_v, vidx_v, ugat_v, vgat_v, uadr_v, vadr_v, out_v,
             usem, vsem):
  wid = lax.axis_index("s") * 2 + lax.axis_index("c")
  base = wid * _BPW

  pltpu.sync_copy(uidx_hbm.at[pl.ds(base, _BPW)], uidx_v)
  pltpu.sync_copy(vidx_hbm.at[pl.ds(base, _BPW)], vidx_v)

  def build(g, carry):
    row0 = pl.multiple_of(g * _K, _K)
    u = uidx_v[pl.ds(row0, _K)]
    v = vidx_v[pl.ds(row0, _K)]
    for c in range(_K):
      r = 2 * g + (c // 8)
      s = (c % 8) * _K
      uadr_v[r, pl.ds(s, _K)] = u + (c * _NUSER)
      vadr_v[r, pl.ds(s, _K)] = v + (c * _NUSER)
    return carry

  lax.fori_loop(0, _NGRP, build, 0)

  copies = []
  for r in range(_NROW):
    copies.append(pltpu.async_copy(
        wf_hbm.at[uadr_v.at[r]], ugat_v.at[r], usem))
    copies.append(pltpu.async_copy(
        hf_hbm.at[vadr_v.at[r]], vgat_v.at[r], vsem))
  for cp in copies:
    cp.wait()

  def dot(g, carry):
    row0 = pl.multiple_of(g * _K, _K)
    acc = jnp.zeros((_K,), jnp.float32)
    for c in range(_K):
      r = 2 * g + (c // 8)
      s = (c % 8) * _K
      acc = acc + ugat_v[r, pl.ds(s, _K)] * vgat_v[r, pl.ds(s, _K)]
    out_v[pl.ds(row0, _K)] = acc
    return carry

  lax.fori_loop(0, _NGRP, dot, 0)

  pltpu.sync_copy(out_v, out_hbm.at[pl.ds(base, _BPW)])


@jax.jit
def kernel(x, W, H):
  uidx = x[:, 0].astype(jnp.int32)
  vidx = x[:, 1].astype(jnp.int32)

  mf = pl.kernel(
      _mf_body,
      out_type=jax.ShapeDtypeStruct((_BATCH,), jnp.float32),
      mesh=plsc.VectorSubcoreMesh(core_axis_name="c", subcore_axis_name="s",
                                  num_cores=2, num_subcores=16),
      compiler_params=pltpu.CompilerParams(needs_layout_passes=False),
      scratch_types=[
          pltpu.VMEM((_BPW,), jnp.int32),
          pltpu.VMEM((_BPW,), jnp.int32),
          pltpu.VMEM((_NROW, 128), jnp.float32),
          pltpu.VMEM((_NROW, 128), jnp.float32),
          pltpu.VMEM((_NROW, 128), jnp.int32),
          pltpu.VMEM((_NROW, 128), jnp.int32),
          pltpu.VMEM((_BPW,), jnp.float32),
          pltpu.SemaphoreType.DMA,
          pltpu.SemaphoreType.DMA,
      ],
  )
  return mf(W.T, H.T, W.T.reshape(-1), H.T.reshape(-1), uidx, vidx)
